# baseline (speedup 1.0000x reference)
import jax
import jax.numpy as jnp
from jax.experimental import pallas as pl
from jax.experimental.pallas import tpu as pltpu

NUM_EMB = 1024
EMB_DIM = 256
BATCH = 9216
TEMP = 0.5
BM = 1024
NGRP = 64

LOG2E = 1.4426950408889634
GBOUND = 16.7


def _vq_block(x_ref, g_ref, t_ref, q_ref, idx_ref, loss_ref,
              esq_ref, taug_ref, ones_ref, grp_ref):
    table = t_ref[...]

    @pl.when(pl.program_id(0) == 0)
    def _():
        esq_ref[...] = jnp.sum(table * table, axis=1)[None, :]
        taug_ref[:, :EMB_DIM] = -2.0 * table
        taug_ref[:, EMB_DIM:2 * EMB_DIM] = table
        taug_ref[:, 2 * EMB_DIM:] = jnp.ones((NUM_EMB, 8), jnp.float32)
        ones_ref[...] = jnp.ones((NUM_EMB, 8), jnp.float32)
        jrow = jax.lax.broadcasted_iota(jnp.int32, (NUM_EMB, NGRP), 0)
        qcol = jax.lax.broadcasted_iota(jnp.int32, (NUM_EMB, NGRP), 1)
        r = jrow - qcol * (NUM_EMB // NGRP)
        pw = jax.lax.bitcast_convert_type(
            jnp.left_shift(NUM_EMB // NGRP - 1 + 127 - r, 23), jnp.float32)
        grp_ref[...] = jnp.where(r == jnp.clip(r, 0, NUM_EMB // NGRP - 1),
                                 pw, 0.0)
        loss_ref[...] = jnp.zeros_like(loss_ref)

    x = x_ref[...]
    xsq = jnp.sum(x * x, axis=1, keepdims=True)
    mm = jax.lax.dot_general(
        x, taug_ref[:, :EMB_DIM], (((1,), (1,)), ((), ())),
        preferred_element_type=jnp.float32)
    d = xsq + esq_ref[...] + mm

    dmin = jnp.min(d, axis=1, keepdims=True)
    t = dmin - d

    mask = jnp.where(t == 0.0, 1.0, 0.0)
    grp = jax.lax.dot_general(
        mask, grp_ref[...], (((1,), (0,)), ((), ())),
        preferred_element_type=jnp.float32)
    ebits = jax.lax.shift_right_logical(
        jax.lax.bitcast_convert_type(grp, jnp.int32), 23)
    qbase = jax.lax.broadcasted_iota(jnp.int32, grp.shape, 1) * (
        NUM_EMB // NGRP) + (NUM_EMB // NGRP - 1 + 127)
    key = (qbase - ebits).astype(jnp.float32)
    idx = jnp.min(jnp.where(grp > 0.0, key, float(NUM_EMB)), axis=1)
    idx_ref[...] = idx.astype(jnp.int32)[None, None, :]

    e1 = jnp.exp2(t * LOG2E)
    e1t = e1 * t
    ones = ones_ref[...]
    s1 = jax.lax.dot_general(
        e1, ones, (((1,), (0,)), ((), ())),
        preferred_element_type=jnp.float32)[:, 0:1]
    s2 = jax.lax.dot_general(
        e1t, ones, (((1,), (0,)), ((), ())),
        preferred_element_type=jnp.float32)[:, 0:1]
    kl_rows = jnp.log(float(NUM_EMB)) - jnp.log(s1) + s2 / s1
    loss_ref[...] += jnp.sum(kl_rows).reshape(1, 1)

    ez = jnp.exp2((g_ref[...] + t) * (2.0 * LOG2E) - (2.0 * LOG2E) * GBOUND)
    qaug = jax.lax.dot_general(
        ez, taug_ref[:, EMB_DIM:], (((1,), (0,)), ((), ())),
        preferred_element_type=jnp.float32)
    sz = qaug[:, EMB_DIM:EMB_DIM + 1]
    q_ref[...] = qaug[:, :EMB_DIM] * (1.0 / sz)


@jax.jit
def kernel(x, var, table, gumbel):
    del var
    nb = BATCH // BM
    q, idx3, loss = pl.pallas_call(
        _vq_block,
        grid=(nb,),
        in_specs=[
            pl.BlockSpec((BM, EMB_DIM), lambda i: (i, 0)),
            pl.BlockSpec((BM, NUM_EMB), lambda i: (i, 0)),
            pl.BlockSpec((NUM_EMB, EMB_DIM), lambda i: (0, 0)),
        ],
        out_specs=[
            pl.BlockSpec((BM, EMB_DIM), lambda i: (i, 0)),
            pl.BlockSpec((1, 1, BM), lambda i: (i, 0, 0)),
            pl.BlockSpec((1, 1), lambda i: (0, 0)),
        ],
        out_shape=[
            jax.ShapeDtypeStruct((BATCH, EMB_DIM), jnp.float32),
            jax.ShapeDtypeStruct((nb, 1, BM), jnp.int32),
            jax.ShapeDtypeStruct((1, 1), jnp.float32),
        ],
        scratch_shapes=[
            pltpu.VMEM((1, NUM_EMB), jnp.float32),
            pltpu.VMEM((NUM_EMB, 2 * EMB_DIM + 8), jnp.float32),
            pltpu.VMEM((NUM_EMB, 8), jnp.float32),
            pltpu.VMEM((NUM_EMB, NGRP), jnp.float32),
        ],
    )(x, gumbel, table)
    return q, loss[0, 0] / BATCH, idx3.reshape(BATCH)


# speedup vs baseline: 1.2569x; 1.2569x over previous
import jax
import jax.numpy as jnp
from jax.experimental import pallas as pl
from jax.experimental.pallas import tpu as pltpu

NUM_EMB = 1024
EMB_DIM = 256
BATCH = 9216
TEMP = 0.5
BM = 1024

LOG2E = 1.4426950408889634
GBOUND = 16.7


def _vq_block(x_ref, g_ref, t_ref, q_ref, idx_ref, loss_ref,
              esq_ref, taug_ref, ones_ref, grp_ref):
    table = t_ref[...]

    @pl.when(pl.program_id(0) == 0)
    def _():
        esq_ref[...] = jnp.sum(table * table, axis=1)[None, :]
        taug_ref[:, :EMB_DIM] = -2.0 * table
        taug_ref[:, EMB_DIM:2 * EMB_DIM] = table
        taug_ref[:, 2 * EMB_DIM:] = jnp.ones((NUM_EMB, 8), jnp.float32)
        ones_ref[...] = jnp.ones((NUM_EMB, 8), jnp.float32)
        grp_ref[...] = jax.lax.broadcasted_iota(
            jnp.int32, (1, NUM_EMB), 1).astype(jnp.float32)
        loss_ref[...] = jnp.zeros_like(loss_ref)

    x = x_ref[...]
    xsq = jnp.sum(x * x, axis=1, keepdims=True)
    mm = jax.lax.dot_general(
        x, taug_ref[:, :EMB_DIM], (((1,), (1,)), ((), ())),
        preferred_element_type=jnp.float32)
    d = xsq + esq_ref[...] + mm

    dmin = jnp.min(d, axis=1, keepdims=True)
    t = dmin - d

    cols = jnp.broadcast_to(grp_ref[...], t.shape)
    idx = jnp.min(jnp.where(t == 0.0, cols, float(NUM_EMB)), axis=1)
    idx_ref[...] = idx.astype(jnp.int32)[None, None, :]

    e1 = jnp.exp2(t * LOG2E)
    e1t = e1 * t
    ones = ones_ref[...]
    s1 = jax.lax.dot_general(
        e1, ones, (((1,), (0,)), ((), ())),
        preferred_element_type=jnp.float32)[:, 0:1]
    s2 = jax.lax.dot_general(
        e1t, ones, (((1,), (0,)), ((), ())),
        preferred_element_type=jnp.float32)[:, 0:1]
    kl_rows = jnp.log(float(NUM_EMB)) - jnp.log(s1) + s2 / s1
    loss_ref[...] += jnp.sum(kl_rows).reshape(1, 1)

    ez = jnp.exp2((g_ref[...] + t) * (2.0 * LOG2E) - (2.0 * LOG2E) * GBOUND)
    qaug = jax.lax.dot_general(
        ez, taug_ref[:, EMB_DIM:], (((1,), (0,)), ((), ())),
        preferred_element_type=jnp.float32)
    sz = qaug[:, EMB_DIM:EMB_DIM + 1]
    q_ref[...] = qaug[:, :EMB_DIM] * (1.0 / sz)


@jax.jit
def kernel(x, var, table, gumbel):
    del var
    nb = BATCH // BM
    q, idx3, loss = pl.pallas_call(
        _vq_block,
        grid=(nb,),
        in_specs=[
            pl.BlockSpec((BM, EMB_DIM), lambda i: (i, 0)),
            pl.BlockSpec((BM, NUM_EMB), lambda i: (i, 0)),
            pl.BlockSpec((NUM_EMB, EMB_DIM), lambda i: (0, 0)),
        ],
        out_specs=[
            pl.BlockSpec((BM, EMB_DIM), lambda i: (i, 0)),
            pl.BlockSpec((1, 1, BM), lambda i: (i, 0, 0)),
            pl.BlockSpec((1, 1), lambda i: (0, 0)),
        ],
        out_shape=[
            jax.ShapeDtypeStruct((BATCH, EMB_DIM), jnp.float32),
            jax.ShapeDtypeStruct((nb, 1, BM), jnp.int32),
            jax.ShapeDtypeStruct((1, 1), jnp.float32),
        ],
        scratch_shapes=[
            pltpu.VMEM((1, NUM_EMB), jnp.float32),
            pltpu.VMEM((NUM_EMB, 2 * EMB_DIM + 8), jnp.float32),
            pltpu.VMEM((NUM_EMB, 8), jnp.float32),
            pltpu.VMEM((1, NUM_EMB), jnp.float32),
        ],
    )(x, gumbel, table)
    return q, loss[0, 0] / BATCH, idx3.reshape(BATCH)


# speedup vs baseline: 1.2574x; 1.0004x over previous
import jax
import jax.numpy as jnp
from jax.experimental import pallas as pl
from jax.experimental.pallas import tpu as pltpu

NUM_EMB = 1024
EMB_DIM = 256
BATCH = 9216
TEMP = 0.5
BM = 1152

LOG2E = 1.4426950408889634
GBOUND = 16.7


def _vq_block(x_ref, g_ref, t_ref, q_ref, idx_ref, loss_ref,
              esq_ref, taug_ref, ones_ref, grp_ref):
    table = t_ref[...]

    @pl.when(pl.program_id(0) == 0)
    def _():
        esq_ref[...] = jnp.sum(table * table, axis=1)[None, :]
        taug_ref[:, :EMB_DIM] = -2.0 * table
        taug_ref[:, EMB_DIM:2 * EMB_DIM] = table
        taug_ref[:, 2 * EMB_DIM:] = jnp.ones((NUM_EMB, 8), jnp.float32)
        ones_ref[...] = jnp.ones((NUM_EMB, 8), jnp.float32)
        grp_ref[...] = jax.lax.broadcasted_iota(
            jnp.int32, (1, NUM_EMB), 1).astype(jnp.float32)
        loss_ref[...] = jnp.zeros_like(loss_ref)

    x = x_ref[...]
    xsq = jnp.sum(x * x, axis=1, keepdims=True)
    mm = jax.lax.dot_general(
        x, taug_ref[:, :EMB_DIM], (((1,), (1,)), ((), ())),
        preferred_element_type=jnp.float32)
    d = xsq + esq_ref[...] + mm

    dmin = jnp.min(d, axis=1, keepdims=True)
    t = dmin - d

    cols = jnp.broadcast_to(grp_ref[...], t.shape)
    idx = jnp.min(jnp.where(t == 0.0, cols, float(NUM_EMB)), axis=1)
    idx_ref[...] = idx.astype(jnp.int32)[None, None, :]

    e1 = jnp.exp2(t * LOG2E)
    e1t = e1 * t
    ones = ones_ref[...]
    s1 = jax.lax.dot_general(
        e1, ones, (((1,), (0,)), ((), ())),
        preferred_element_type=jnp.float32)[:, 0:1]
    s2 = jax.lax.dot_general(
        e1t, ones, (((1,), (0,)), ((), ())),
        preferred_element_type=jnp.float32)[:, 0:1]
    kl_rows = jnp.log(float(NUM_EMB)) - jnp.log(s1) + s2 / s1
    loss_ref[...] += jnp.sum(kl_rows).reshape(1, 1)

    ez = jnp.exp2((g_ref[...] + t) * (2.0 * LOG2E) - (2.0 * LOG2E) * GBOUND)
    qaug = jax.lax.dot_general(
        ez, taug_ref[:, EMB_DIM:], (((1,), (0,)), ((), ())),
        preferred_element_type=jnp.float32)
    sz = qaug[:, EMB_DIM:EMB_DIM + 1]
    q_ref[...] = qaug[:, :EMB_DIM] * (1.0 / sz)


@jax.jit
def kernel(x, var, table, gumbel):
    del var
    nb = BATCH // BM
    q, idx3, loss = pl.pallas_call(
        _vq_block,
        grid=(nb,),
        in_specs=[
            pl.BlockSpec((BM, EMB_DIM), lambda i: (i, 0)),
            pl.BlockSpec((BM, NUM_EMB), lambda i: (i, 0)),
            pl.BlockSpec((NUM_EMB, EMB_DIM), lambda i: (0, 0)),
        ],
        out_specs=[
            pl.BlockSpec((BM, EMB_DIM), lambda i: (i, 0)),
            pl.BlockSpec((1, 1, BM), lambda i: (i, 0, 0)),
            pl.BlockSpec((1, 1), lambda i: (0, 0)),
        ],
        out_shape=[
            jax.ShapeDtypeStruct((BATCH, EMB_DIM), jnp.float32),
            jax.ShapeDtypeStruct((nb, 1, BM), jnp.int32),
            jax.ShapeDtypeStruct((1, 1), jnp.float32),
        ],
        scratch_shapes=[
            pltpu.VMEM((1, NUM_EMB), jnp.float32),
            pltpu.VMEM((NUM_EMB, 2 * EMB_DIM + 8), jnp.float32),
            pltpu.VMEM((NUM_EMB, 8), jnp.float32),
            pltpu.VMEM((1, NUM_EMB), jnp.float32),
        ],
    )(x, gumbel, table)
    return q, loss[0, 0] / BATCH, idx3.reshape(BATCH)


# speedup vs baseline: 1.2583x; 1.0007x over previous
import jax
import jax.numpy as jnp
from jax.experimental import pallas as pl
from jax.experimental.pallas import tpu as pltpu

NUM_EMB = 1024
EMB_DIM = 256
BATCH = 9216
TEMP = 0.5
BM = 1024

LOG2E = 1.4426950408889634
GBOUND = 16.7


def _vq_block(x_ref, g_ref, t_ref, q_ref, idx_ref, loss_ref,
              esq_ref, taug_ref, ones_ref, grp_ref):
    table = t_ref[...]

    @pl.when(pl.program_id(0) == 0)
    def _():
        esq_ref[...] = jnp.sum(table * table, axis=1)[None, :]
        taug_ref[:, :EMB_DIM] = -2.0 * table
        taug_ref[:, EMB_DIM:2 * EMB_DIM] = table
        taug_ref[:, 2 * EMB_DIM:] = jnp.ones((NUM_EMB, 8), jnp.float32)
        ones_ref[...] = jnp.ones((NUM_EMB, 8), jnp.float32)
        grp_ref[...] = jax.lax.broadcasted_iota(
            jnp.int32, (1, NUM_EMB), 1).astype(jnp.float32)
        loss_ref[...] = jnp.zeros_like(loss_ref)

    x = x_ref[...]
    xsq = jnp.sum(x * x, axis=1, keepdims=True)
    mm = jax.lax.dot_general(
        x, taug_ref[:, :EMB_DIM], (((1,), (1,)), ((), ())),
        preferred_element_type=jnp.float32)
    d = xsq + esq_ref[...] + mm

    dmin = jnp.min(d, axis=1, keepdims=True)
    t = dmin - d

    cols = jnp.broadcast_to(grp_ref[...], t.shape)
    idx = jnp.min(jnp.where(t == 0.0, cols, float(NUM_EMB)), axis=1)
    idx_ref[...] = idx.astype(jnp.int32)[None, None, :]

    e1 = jnp.exp2(t * LOG2E)
    e1t = e1 * t
    ones = ones_ref[...]
    s1 = jax.lax.dot_general(
        e1, ones, (((1,), (0,)), ((), ())),
        preferred_element_type=jnp.float32)[:, 0:1]
    s2 = jax.lax.dot_general(
        e1t, ones, (((1,), (0,)), ((), ())),
        preferred_element_type=jnp.float32)[:, 0:1]
    kl_rows = jnp.log(float(NUM_EMB)) - jnp.log(s1) + s2 / s1
    loss_ref[...] += jnp.sum(kl_rows).reshape(1, 1)

    ez = jnp.exp2((g_ref[...] + t) * (2.0 * LOG2E) - (2.0 * LOG2E) * GBOUND)
    qaug = jax.lax.dot_general(
        ez, taug_ref[:, EMB_DIM:], (((1,), (0,)), ((), ())),
        preferred_element_type=jnp.float32)
    sz = qaug[:, EMB_DIM:EMB_DIM + 1]
    q_ref[...] = qaug[:, :EMB_DIM] * (1.0 / sz)


@jax.jit
def kernel(x, var, table, gumbel):
    del var
    nb = BATCH // BM
    q, idx3, loss = pl.pallas_call(
        _vq_block,
        grid=(nb,),
        in_specs=[
            pl.BlockSpec((BM, EMB_DIM), lambda i: (i, 0)),
            pl.BlockSpec((BM, NUM_EMB), lambda i: (i, 0)),
            pl.BlockSpec((NUM_EMB, EMB_DIM), lambda i: (0, 0)),
        ],
        out_specs=[
            pl.BlockSpec((BM, EMB_DIM), lambda i: (i, 0)),
            pl.BlockSpec((1, 1, BM), lambda i: (i, 0, 0)),
            pl.BlockSpec((1, 1), lambda i: (0, 0)),
        ],
        out_shape=[
            jax.ShapeDtypeStruct((BATCH, EMB_DIM), jnp.float32),
            jax.ShapeDtypeStruct((nb, 1, BM), jnp.int32),
            jax.ShapeDtypeStruct((1, 1), jnp.float32),
        ],
        scratch_shapes=[
            pltpu.VMEM((1, NUM_EMB), jnp.float32),
            pltpu.VMEM((NUM_EMB, 2 * EMB_DIM + 8), jnp.float32),
            pltpu.VMEM((NUM_EMB, 8), jnp.float32),
            pltpu.VMEM((1, NUM_EMB), jnp.float32),
        ],
    )(x, gumbel, table)
    return q, loss[0, 0] / BATCH, idx3.reshape(BATCH)
